# baseline (speedup 1.0000x reference)
import jax
import jax.numpy as jnp
from jax import lax
from jax.experimental import pallas as pl
from jax.experimental.pallas import tpu as pltpu
from jax.experimental.pallas import tpu_sc as plsc

N = 10000
D = 128
NC = 2
NS = 16
NW = NC * NS
B = 128
NP = 10240
RPS = NP // NS
NPD = 10240
RPSD = NPD // NS
F0 = 0.72

_mesh = plsc.VectorSubcoreMesh(
    core_axis_name="c", subcore_axis_name="s", num_cores=NC, num_subcores=NS)


def _worker_ids():
    c = lax.axis_index("c")
    s = lax.axis_index("s")
    return c, s, s * NC + c


def _degree_body(dstp_hbm, zeros1_hbm, out_hbm, dst_v, ones_v, deg_sh, sem):
    c, s, w = _worker_ids()
    pltpu.sync_copy(zeros1_hbm, deg_sh.at[pl.ds(s * RPSD, RPSD)])
    pltpu.sync_copy(dstp_hbm.at[w], dst_v)
    for g in range(B // 16):
        ones_v[pl.ds(g * 16, 16)] = jnp.ones((16,), jnp.float32)
    plsc.subcore_barrier()

    def step(j, _):
        pltpu.sync_copy(ones_v, deg_sh.at[dst_v.at[j]], add=True)
        return 0

    lax.fori_loop(0, dstp_hbm.shape[1], step, 0)
    plsc.subcore_barrier()
    pltpu.sync_copy(deg_sh.at[pl.ds(s * RPSD, RPSD)],
                    out_hbm.at[pl.ds(c * NPD + s * RPSD, RPSD)])


def _degree(dstp, zeros1, cwd):
    kern = pl.kernel(
        _degree_body,
        out_type=jax.ShapeDtypeStruct((NC * NPD,), jnp.float32),
        mesh=_mesh,
        scratch_types=[
            pltpu.VMEM((cwd, B), jnp.int32),
            pltpu.VMEM((B,), jnp.float32),
            pltpu.VMEM_SHARED((NPD,), jnp.float32),
            pltpu.SemaphoreType.DMA,
        ],
    )
    return kern(dstp, zeros1)


def _agg_body(y_hbm, srcp_hbm, dstp_hbm, zeros2_hbm, out_hbm,
              src_v, dst_v, rows_v, acc_sh, sem):
    c, s, w = _worker_ids()
    h = srcp_hbm.shape[1] // 2
    pltpu.sync_copy(zeros2_hbm, acc_sh.at[pl.ds(s * RPS, RPS)])
    plsc.subcore_barrier()

    def step(i, _):
        a = 2 * i
        g0 = pltpu.async_copy(y_hbm.at[src_v.at[a]], rows_v.at[0], sem)
        g1 = pltpu.async_copy(y_hbm.at[src_v.at[a + 1]], rows_v.at[1], sem)
        g0.wait()
        pltpu.sync_copy(rows_v.at[0], acc_sh.at[dst_v.at[a]], add=True)
        g1.wait()
        pltpu.sync_copy(rows_v.at[1], acc_sh.at[dst_v.at[a + 1]], add=True)
        return 0

    for half in range(2):
        pltpu.sync_copy(srcp_hbm.at[w, pl.ds(half * h, h)], src_v)
        pltpu.sync_copy(dstp_hbm.at[w, pl.ds(half * h, h)], dst_v)
        lax.fori_loop(0, h // 2, step, 0)

    plsc.subcore_barrier()
    pltpu.sync_copy(acc_sh.at[pl.ds(s * RPS, RPS)],
                    out_hbm.at[c, pl.ds(s * RPS, RPS)])


def _aggregate(y, srcp, dstp, zeros2):
    kern = pl.kernel(
        _agg_body,
        out_type=jax.ShapeDtypeStruct((NC, NP, D), jnp.float32),
        mesh=_mesh,
        scratch_types=[
            pltpu.VMEM((srcp.shape[1] // 2, B), jnp.int32),
            pltpu.VMEM((srcp.shape[1] // 2, B), jnp.int32),
            pltpu.VMEM((2, B, D), jnp.float32),
            pltpu.VMEM_SHARED((NP, D), jnp.float32),
            pltpu.SemaphoreType.DMA,
        ],
    )
    return kern(y, srcp, dstp, zeros2)


def _tc1_body(x_ref, w_ref, degt_ref, y_ref, dinv_ref):
    deg = 1.0 + degt_ref[:, 0:1] + degt_ref[:, 1:2]
    dinv = lax.rsqrt(deg)
    dinv_ref[...] = dinv
    xw = jnp.dot(x_ref[...], w_ref[...], preferred_element_type=jnp.float32)
    y_ref[...] = xw * dinv


def _tc1(x_pad, W1, degt):
    return pl.pallas_call(
        _tc1_body,
        out_shape=(jax.ShapeDtypeStruct((NP, D), jnp.float32),
                   jax.ShapeDtypeStruct((NP, 1), jnp.float32)),
    )(x_pad, W1, degt)


def _tc2_body(p_ref, y_ref, dinv_ref, b_ref, w_ref, y2_ref):
    dinv = dinv_ref[...]
    pre = dinv * (p_ref[0] + p_ref[1] + y_ref[...]) + b_ref[...]
    h = jnp.maximum(pre, 0.0)
    y2_ref[...] = jnp.dot(h, w_ref[...],
                          preferred_element_type=jnp.float32) * dinv


def _tc2(p, y1, dinv, b1, W2):
    return pl.pallas_call(
        _tc2_body,
        out_shape=jax.ShapeDtypeStruct((NP, D), jnp.float32),
    )(p, y1, dinv, b1, W2)


def _tc3_body(p_ref, y_ref, dinv_ref, b_ref, out_ref):
    out_ref[...] = dinv_ref[...] * (p_ref[0] + p_ref[1] + y_ref[...]) \
        + b_ref[...]


def _tc3(p, y2, dinv, b2):
    return pl.pallas_call(
        _tc3_body,
        out_shape=jax.ShapeDtypeStruct((NP, D), jnp.float32),
    )(p, y2, dinv, b2)


@jax.jit
def kernel(x, edge_index, batch, W1, b1, W2, b2):
    del batch
    x = x.astype(jnp.float32)
    e = edge_index.shape[1]
    src = edge_index[0].astype(jnp.int32)
    dst = edge_index[1].astype(jnp.int32)

    cwd = -(-e // (NW * B))
    epd = NW * B * cwd
    dst_pd = jnp.concatenate(
        [dst, jnp.full((epd - e,), N, jnp.int32)]).reshape(NW, cwd, B)

    cw = -(-e // (NW * B))
    cw = -(-cw // 4) * 4
    ep = NW * B * cw
    src_p = jnp.concatenate(
        [src, jnp.zeros((ep - e,), jnp.int32)]).reshape(NW, cw, B)
    dst_p = jnp.concatenate(
        [dst, jnp.full((ep - e,), N, jnp.int32)]).reshape(NW, cw, B)
    x_pad = jnp.zeros((NP, D), jnp.float32).at[:N].set(x)
    zeros1 = jnp.zeros((RPSD,), jnp.float32)
    zeros2 = jnp.zeros((RPS, D), jnp.float32)

    degp = _degree(dst_pd, zeros1, cwd).reshape(NC, NPD)[:, :NP]
    degt = jnp.transpose(degp)
    y1, dinv = _tc1(x_pad, W1, degt)
    p1 = _aggregate(y1, src_p, dst_p, zeros2)
    y2 = _tc2(p1, y1, dinv, b1.reshape(1, D), W2)
    p2 = _aggregate(y2, src_p, dst_p, zeros2)
    out = _tc3(p2, y2, dinv, b2.reshape(1, D))
    return out[:N]


# speedup vs baseline: 1.2742x; 1.2742x over previous
import jax
import jax.numpy as jnp
from jax import lax
from jax.experimental import pallas as pl
from jax.experimental.pallas import tpu as pltpu
from jax.experimental.pallas import tpu_sc as plsc

N = 10000
D = 128
NC = 2
NS = 16
NW = NC * NS
B = 128
NP = 10240
RPS = NP // NS
NPD = 10240
RPSD = NPD // NS
F0 = 0.72

_mesh = plsc.VectorSubcoreMesh(
    core_axis_name="c", subcore_axis_name="s", num_cores=NC, num_subcores=NS)


def _worker_ids():
    c = lax.axis_index("c")
    s = lax.axis_index("s")
    return c, s, s * NC + c


def _degree_body(dstp_hbm, zeros1_hbm, out_hbm, dst_v, ones_v, deg_sh, sem):
    c, s, w = _worker_ids()
    pltpu.sync_copy(zeros1_hbm, deg_sh.at[pl.ds(s * RPSD, RPSD)])
    pltpu.sync_copy(dstp_hbm.at[w], dst_v)
    for g in range(B // 16):
        ones_v[pl.ds(g * 16, 16)] = jnp.ones((16,), jnp.float32)
    plsc.subcore_barrier()

    def step(j, _):
        pltpu.sync_copy(ones_v, deg_sh.at[dst_v.at[j]], add=True)
        return 0

    lax.fori_loop(0, dstp_hbm.shape[1], step, 0)
    plsc.subcore_barrier()
    pltpu.sync_copy(deg_sh.at[pl.ds(s * RPSD, RPSD)],
                    out_hbm.at[pl.ds(c * NPD + s * RPSD, RPSD)])


def _degree(dstp, zeros1, cwd):
    kern = pl.kernel(
        _degree_body,
        out_type=jax.ShapeDtypeStruct((NC * NPD,), jnp.float32),
        mesh=_mesh,
        scratch_types=[
            pltpu.VMEM((cwd, B), jnp.int32),
            pltpu.VMEM((B,), jnp.float32),
            pltpu.VMEM_SHARED((NPD,), jnp.float32),
            pltpu.SemaphoreType.DMA,
        ],
    )
    return kern(dstp, zeros1)


def _agg_body(y_hbm, srcp_hbm, dstp_hbm, zeros2_hbm, out_hbm,
              src_v, dst_v, rows_v, acc_sh, sem):
    c, s, w = _worker_ids()
    pltpu.sync_copy(zeros2_hbm, acc_sh.at[pl.ds(s * RPS, RPS)])
    pltpu.sync_copy(srcp_hbm.at[w], src_v)
    pltpu.sync_copy(dstp_hbm.at[w], dst_v)
    plsc.subcore_barrier()

    def step(j, _):
        pltpu.async_copy(y_hbm.at[src_v.at[j]], rows_v, sem).wait()
        pltpu.sync_copy(rows_v, acc_sh.at[dst_v.at[j]], add=True)
        return 0

    lax.fori_loop(0, srcp_hbm.shape[1], step, 0)
    plsc.subcore_barrier()
    pltpu.sync_copy(acc_sh.at[pl.ds(s * RPS, RPS)],
                    out_hbm.at[c, pl.ds(s * RPS, RPS)])


def _aggregate(y, srcp, dstp, zeros2):
    kern = pl.kernel(
        _agg_body,
        out_type=jax.ShapeDtypeStruct((NC, NP, D), jnp.float32),
        mesh=_mesh,
        scratch_types=[
            pltpu.VMEM((srcp.shape[1], B), jnp.int32),
            pltpu.VMEM((srcp.shape[1], B), jnp.int32),
            pltpu.VMEM((B, D), jnp.float32),
            pltpu.VMEM_SHARED((NP, D), jnp.float32),
            pltpu.SemaphoreType.DMA,
        ],
    )
    return kern(y, srcp, dstp, zeros2)


def _tc1_body(x_ref, w_ref, degt_ref, y_ref, dinv_ref):
    deg = 1.0 + degt_ref[:, 0:1] + degt_ref[:, 1:2]
    dinv = lax.rsqrt(deg)
    dinv_ref[...] = dinv
    xw = jnp.dot(x_ref[...], w_ref[...], preferred_element_type=jnp.float32)
    y_ref[...] = xw * dinv


def _tc1(x_pad, W1, degt):
    return pl.pallas_call(
        _tc1_body,
        out_shape=(jax.ShapeDtypeStruct((NP, D), jnp.float32),
                   jax.ShapeDtypeStruct((NP, 1), jnp.float32)),
    )(x_pad, W1, degt)


def _tc2_body(p_ref, y_ref, dinv_ref, b_ref, w_ref, y2_ref):
    dinv = dinv_ref[...]
    pre = dinv * (p_ref[0] + p_ref[1] + y_ref[...]) + b_ref[...]
    h = jnp.maximum(pre, 0.0)
    y2_ref[...] = jnp.dot(h, w_ref[...],
                          preferred_element_type=jnp.float32) * dinv


def _tc2(p, y1, dinv, b1, W2):
    return pl.pallas_call(
        _tc2_body,
        out_shape=jax.ShapeDtypeStruct((NP, D), jnp.float32),
    )(p, y1, dinv, b1, W2)


def _tc3_body(p_ref, y_ref, dinv_ref, b_ref, out_ref):
    out_ref[...] = dinv_ref[...] * (p_ref[0] + p_ref[1] + y_ref[...]) \
        + b_ref[...]


def _tc3(p, y2, dinv, b2):
    return pl.pallas_call(
        _tc3_body,
        out_shape=jax.ShapeDtypeStruct((NP, D), jnp.float32),
    )(p, y2, dinv, b2)


@jax.jit
def kernel(x, edge_index, batch, W1, b1, W2, b2):
    del batch
    x = x.astype(jnp.float32)
    e = edge_index.shape[1]
    src = edge_index[0].astype(jnp.int32)
    dst = edge_index[1].astype(jnp.int32)

    cwd = -(-e // (NW * B))
    epd = NW * B * cwd
    dst_pd = jnp.concatenate(
        [dst, jnp.full((epd - e,), N, jnp.int32)]).reshape(NW, cwd, B)

    cw = -(-e // (NW * B))
    ep = NW * B * cw
    src_p = jnp.concatenate(
        [src, jnp.zeros((ep - e,), jnp.int32)]).reshape(NW, cw, B)
    dst_p = jnp.concatenate(
        [dst, jnp.full((ep - e,), N, jnp.int32)]).reshape(NW, cw, B)
    x_pad = jnp.zeros((NP, D), jnp.float32).at[:N].set(x)
    zeros1 = jnp.zeros((RPSD,), jnp.float32)
    zeros2 = jnp.zeros((RPS, D), jnp.float32)

    degp = _degree(dst_pd, zeros1, cwd).reshape(NC, NPD)[:, :NP]
    degt = jnp.transpose(degp)
    y1, dinv = _tc1(x_pad, W1, degt)
    p1 = _aggregate(y1, src_p, dst_p, zeros2)
    y2 = _tc2(p1, y1, dinv, b1.reshape(1, D), W2)
    p2 = _aggregate(y2, src_p, dst_p, zeros2)
    out = _tc3(p2, y2, dinv, b2.reshape(1, D))
    return out[:N]


# speedup vs baseline: 1.2749x; 1.0006x over previous
import jax
import jax.numpy as jnp
from jax import lax
from jax.experimental import pallas as pl
from jax.experimental.pallas import tpu as pltpu
from jax.experimental.pallas import tpu_sc as plsc

N = 10000
D = 128
NC = 2
NS = 16
NW = NC * NS
B = 128
NP = 10240
RPS = NP // NS
NPD = 10240
RPSD = NPD // NS


def _mesh():
    return plsc.VectorSubcoreMesh(
        core_axis_name="c", subcore_axis_name="s",
        num_cores=NC, num_subcores=NS)


def _worker_ids():
    c = lax.axis_index("c")
    s = lax.axis_index("s")
    return c, s, s * NC + c


def _degree_body(dstp_hbm, zeros1_hbm, out_hbm, dst_v, ones_v, deg_sh, sem):
    c, s, w = _worker_ids()
    pltpu.sync_copy(zeros1_hbm, deg_sh.at[pl.ds(s * RPSD, RPSD)])
    pltpu.sync_copy(dstp_hbm.at[w], dst_v)
    for g in range(B // 16):
        ones_v[pl.ds(g * 16, 16)] = jnp.ones((16,), jnp.float32)
    plsc.subcore_barrier()

    def step(j, _):
        pltpu.sync_copy(ones_v, deg_sh.at[dst_v.at[j]], add=True)
        return 0

    lax.fori_loop(0, dstp_hbm.shape[1], step, 0)
    plsc.subcore_barrier()
    pltpu.sync_copy(deg_sh.at[pl.ds(s * RPSD, RPSD)],
                    out_hbm.at[pl.ds(c * NPD + s * RPSD, RPSD)])


def _degree(dstp, zeros1, cwd):
    kern = pl.kernel(
        _degree_body,
        out_type=jax.ShapeDtypeStruct((NC * NPD,), jnp.float32),
        mesh=_mesh(),
        scratch_types=[
            pltpu.VMEM((cwd, B), jnp.int32),
            pltpu.VMEM((B,), jnp.float32),
            pltpu.VMEM_SHARED((NPD,), jnp.float32),
            pltpu.SemaphoreType.DMA,
        ],
    )
    return kern(dstp, zeros1)


def _agg_body(y_hbm, srcp_hbm, dstp_hbm, zeros2_hbm, out_hbm,
              src_v, dst_v, rows_v, acc_sh, sem):
    c, s, w = _worker_ids()
    pltpu.sync_copy(zeros2_hbm, acc_sh.at[pl.ds(s * RPS, RPS)])
    pltpu.sync_copy(srcp_hbm.at[w], src_v)
    pltpu.sync_copy(dstp_hbm.at[w], dst_v)
    plsc.subcore_barrier()

    def step(j, _):
        pltpu.async_copy(y_hbm.at[src_v.at[j]], rows_v, sem).wait()
        pltpu.sync_copy(rows_v, acc_sh.at[dst_v.at[j]], add=True)
        return 0

    lax.fori_loop(0, srcp_hbm.shape[1], step, 0)
    plsc.subcore_barrier()
    pltpu.sync_copy(acc_sh.at[pl.ds(s * RPS, RPS)],
                    out_hbm.at[c, pl.ds(s * RPS, RPS)])


def _aggregate(y, srcp, dstp, zeros2):
    kern = pl.kernel(
        _agg_body,
        out_type=jax.ShapeDtypeStruct((NC, NP, D), jnp.float32),
        mesh=_mesh(),
        scratch_types=[
            pltpu.VMEM((srcp.shape[1], B), jnp.int32),
            pltpu.VMEM((srcp.shape[1], B), jnp.int32),
            pltpu.VMEM((B, D), jnp.float32),
            pltpu.VMEM_SHARED((NP, D), jnp.float32),
            pltpu.SemaphoreType.DMA,
        ],
    )
    return kern(y, srcp, dstp, zeros2)


def _tc1_body(x_ref, w_ref, degt_ref, y_ref, dinv_ref):
    deg = 1.0 + degt_ref[:, 0:1] + degt_ref[:, 1:2]
    dinv = lax.rsqrt(deg)
    dinv_ref[...] = dinv
    xw = jnp.dot(x_ref[...], w_ref[...], preferred_element_type=jnp.float32)
    y_ref[...] = xw * dinv


def _tc1(x_pad, W1, degt):
    return pl.pallas_call(
        _tc1_body,
        out_shape=(jax.ShapeDtypeStruct((NP, D), jnp.float32),
                   jax.ShapeDtypeStruct((NP, 1), jnp.float32)),
    )(x_pad, W1, degt)


def _tc2_body(p_ref, y_ref, dinv_ref, b_ref, w_ref, y2_ref):
    dinv = dinv_ref[...]
    pre = dinv * (p_ref[0] + p_ref[1] + y_ref[...]) + b_ref[...]
    h = jnp.maximum(pre, 0.0)
    y2_ref[...] = jnp.dot(h, w_ref[...],
                          preferred_element_type=jnp.float32) * dinv


def _tc2(p, y1, dinv, b1, W2):
    return pl.pallas_call(
        _tc2_body,
        out_shape=jax.ShapeDtypeStruct((NP, D), jnp.float32),
    )(p, y1, dinv, b1, W2)


def _tc3_body(p_ref, y_ref, dinv_ref, b_ref, out_ref):
    out_ref[...] = dinv_ref[...] * (p_ref[0] + p_ref[1] + y_ref[...]) \
        + b_ref[...]


def _tc3(p, y2, dinv, b2):
    return pl.pallas_call(
        _tc3_body,
        out_shape=jax.ShapeDtypeStruct((NP, D), jnp.float32),
    )(p, y2, dinv, b2)


@jax.jit
def kernel(x, edge_index, batch, W1, b1, W2, b2):
    del batch
    x = x.astype(jnp.float32)
    e = edge_index.shape[1]
    src = edge_index[0].astype(jnp.int32)
    dst = edge_index[1].astype(jnp.int32)

    cwd = -(-e // (NW * B))
    epd = NW * B * cwd
    dst_pd = jnp.concatenate(
        [dst, jnp.full((epd - e,), N, jnp.int32)]).reshape(NW, cwd, B)

    cw = -(-e // (NW * B))
    ep = NW * B * cw
    src_p = jnp.concatenate(
        [src, jnp.zeros((ep - e,), jnp.int32)]).reshape(NW, cw, B)
    dst_p = jnp.concatenate(
        [dst, jnp.full((ep - e,), N, jnp.int32)]).reshape(NW, cw, B)
    x_pad = jnp.zeros((NP, D), jnp.float32).at[:N].set(x)
    zeros1 = jnp.zeros((RPSD,), jnp.float32)
    zeros2 = jnp.zeros((RPS, D), jnp.float32)

    degp = _degree(dst_pd, zeros1, cwd).reshape(NC, NPD)[:, :NP]
    degt = jnp.transpose(degp)
    y1, dinv = _tc1(x_pad, W1, degt)
    p1 = _aggregate(y1, src_p, dst_p, zeros2)
    y2 = _tc2(p1, y1, dinv, b1.reshape(1, D), W2)
    p2 = _aggregate(y2, src_p, dst_p, zeros2)
    out = _tc3(p2, y2, dinv, b2.reshape(1, D))
    return out[:N]
